# baseline (speedup 1.0000x reference)
import functools

import jax
import jax.numpy as jnp
from jax import lax
from jax.experimental import pallas as pl
from jax.experimental.pallas import tpu as pltpu
from jax.experimental.pallas import tpu_sc as plsc

NC = 2
NS = 16
NW = NC * NS
L = 16
C = 4
R = 4


def _extract(vec_ref, j):
    grp = (j // L) * L
    vec = vec_ref[pl.ds(grp, L)]
    onehot = lax.iota(jnp.int32, L) == (j - grp)
    return jnp.sum(jnp.where(onehot, vec, 0))


@functools.partial(jax.jit, static_argnums=(4, 5))
def _sc_masked_add(x2, a2, gidx, starts, N, D):
    RW = N // NW
    nchunk = RW // C

    def body(x_hbm, a_hbm, gidx_hbm, starts_hbm, out_hbm,
             buf, abuf, idxv, stv,
             in0, in1, in2, in3, ou0, ou1, ou2, ou3, gsem):
        ins = (in0, in1, in2, in3)
        outs = (ou0, ou1, ou2, ou3)
        cid = lax.axis_index("c")
        sid = lax.axis_index("s")
        w = sid * NC + cid
        base = w * RW
        pltpu.sync_copy(gidx_hbm.at[w], idxv)
        pltpu.sync_copy(starts_hbm.at[w], stv)

        def issue_in(c, t):
            pltpu.async_copy(x_hbm.at[pl.ds(base + c * C, C)],
                             buf.at[t], ins[t])

        issue_in(0, 0)

        def chunk_work(c, u, s):
            t = u % R
            e = _extract(stv, c + 1)
            k = e - s

            def fire(i, _):
                g = _extract(idxv, s + i)
                pltpu.async_copy(a_hbm.at[pl.ds(g, 1)],
                                 abuf.at[pl.ds(i, 1)], gsem)
                return _

            lax.fori_loop(0, k, fire, 0)
            pltpu.make_async_copy(x_hbm.at[pl.ds(base, C)],
                                  buf.at[t], ins[t]).wait()
            t1 = (u + 1) % R

            @pl.when(c + 1 < nchunk)
            def _():
                @pl.when(c >= R - 1)
                def _():
                    pltpu.make_async_copy(buf.at[t1],
                                          out_hbm.at[pl.ds(base, C)],
                                          outs[t1]).wait()
                issue_in(c + 1, t1)

            def drain(i, _):
                pltpu.make_async_copy(a_hbm.at[pl.ds(base, 1)],
                                      abuf.at[pl.ds(0, 1)], gsem).wait()
                return _

            lax.fori_loop(0, k, drain, 0)

            def patch(i, _):
                g = _extract(idxv, s + i)
                p = g - (base + c * C)

                def add_body(d, c2):
                    slc = pl.ds(d * L, L)
                    plsc.addupdate(buf.at[t, p, slc], abuf[i, slc])
                    return c2

                lax.fori_loop(0, D // L, add_body, 0, unroll=8)
                return _

            lax.fori_loop(0, k, patch, 0)
            pltpu.async_copy(buf.at[t], out_hbm.at[pl.ds(base + c * C, C)],
                             outs[t])
            return e

        def group_body(gi, s):
            for u in range(R):
                s = chunk_work(gi * R + u, u, s)
            return s

        lax.fori_loop(0, nchunk // R, group_body, 0)
        for t in range(R):
            pltpu.make_async_copy(buf.at[t], out_hbm.at[pl.ds(base, C)],
                                  outs[t]).wait()

    fn = pl.kernel(
        body,
        out_type=jax.ShapeDtypeStruct((N, D), jnp.float32),
        mesh=plsc.VectorSubcoreMesh(
            core_axis_name="c", subcore_axis_name="s",
            num_cores=NC, num_subcores=NS),
        scratch_types=[
            pltpu.VMEM((R, C, D), jnp.float32),
            pltpu.VMEM((C, D), jnp.float32),
            pltpu.VMEM((RW,), jnp.int32),
            pltpu.VMEM((96,), jnp.int32),
        ] + [pltpu.SemaphoreType.DMA] * 9,
        compiler_params=pltpu.CompilerParams(needs_layout_passes=False),
    )
    return fn(x2, a2, gidx, starts)


K_SC = 4096


def _tc_body(x_ref, a_ref, m_ref, o_ref):
    m = m_ref[...]
    o_ref[...] = jnp.where(m != 0.0, x_ref[...] + a_ref[...], x_ref[...])


def _tc_where(x2, a2, mf, K, N, D):
    RB = 256
    off = K // RB
    return pl.pallas_call(
        _tc_body,
        grid=((N - K) // RB,),
        in_specs=[
            pl.BlockSpec((RB, D), lambda i: (i + off, 0)),
            pl.BlockSpec((RB, D), lambda i: (i + off, 0)),
            pl.BlockSpec((RB, 1), lambda i: (i + off, 0)),
        ],
        out_specs=pl.BlockSpec((RB, D), lambda i: (i, 0)),
        out_shape=jax.ShapeDtypeStruct((N - K, D), jnp.float32),
    )(x2, a2, mf)


def kernel(x, attack, attack_mask):
    B, S, D = x.shape
    N = B * S
    K = K_SC
    RW = K // NW
    x2 = x.reshape(N, D)
    a2 = attack.astype(x.dtype).reshape(N, D)
    m = attack_mask[:, :S].reshape(N)
    m2 = m[:K].reshape(NW, RW)
    loc = jnp.argsort(~m2, axis=1, stable=True).astype(jnp.int32)
    gidx = loc + (jnp.arange(NW, dtype=jnp.int32) * RW)[:, None]
    ccnt = m2.reshape(NW, RW // C, C).sum(-1).astype(jnp.int32)
    starts = jnp.concatenate(
        [jnp.zeros((NW, 1), jnp.int32), jnp.cumsum(ccnt, axis=1)], axis=1)
    starts = jnp.pad(starts, ((0, 0), (0, 96 - starts.shape[1])))
    sc_out = _sc_masked_add(x2, a2, gidx, starts, K, D)
    if K < N:
        mf = m.astype(jnp.float32)[:, None]
        tc_out = _tc_where(x2, a2, mf, K, N, D)
        out2 = jnp.concatenate([sc_out, tc_out], axis=0)
    else:
        out2 = sc_out
    return out2.reshape(B, S, D)


# speedup vs baseline: 1.2174x; 1.2174x over previous
import functools

import jax
import jax.numpy as jnp
from jax import lax
from jax.experimental import pallas as pl
from jax.experimental.pallas import tpu as pltpu
from jax.experimental.pallas import tpu_sc as plsc

NC = 2
NS = 16
NW = NC * NS
L = 16
C = 4
R = 4


def _extract(vec_ref, j):
    grp = (j // L) * L
    vec = vec_ref[pl.ds(grp, L)]
    onehot = lax.iota(jnp.int32, L) == (j - grp)
    return jnp.sum(jnp.where(onehot, vec, 0))


@functools.partial(jax.jit, static_argnums=(4, 5))
def _sc_masked_add(x2, a2, gidx, starts, N, D):
    RW = N // NW
    nchunk = RW // C

    def body(x_hbm, a_hbm, gidx_hbm, starts_hbm, out_hbm,
             buf, abuf, idxv, stv,
             in0, in1, in2, in3, ou0, ou1, ou2, ou3, gs0, gs1):
        ins = (in0, in1, in2, in3)
        outs = (ou0, ou1, ou2, ou3)
        gss = (gs0, gs1)
        cid = lax.axis_index("c")
        sid = lax.axis_index("s")
        w = sid * NC + cid
        base = w * RW
        pltpu.sync_copy(gidx_hbm.at[w], idxv)
        pltpu.sync_copy(starts_hbm.at[w], stv)

        def issue_in(c, t):
            pltpu.async_copy(x_hbm.at[pl.ds(base + c * C, C)],
                             buf.at[t], ins[t])

        def fire_rows(s, e, q):
            def fire(i, _):
                g = _extract(idxv, s + i)
                pltpu.async_copy(a_hbm.at[pl.ds(g, 1)],
                                 abuf.at[q, pl.ds(i, 1)], gss[q])
                return _

            lax.fori_loop(0, e - s, fire, 0)

        issue_in(0, 0)
        e0 = _extract(stv, 1)
        fire_rows(0, e0, 0)

        def chunk_work(c, u, carry):
            s, e = carry
            t = u % R
            q = u % 2
            k = e - s
            pltpu.make_async_copy(x_hbm.at[pl.ds(base, C)],
                                  buf.at[t], ins[t]).wait()
            t1 = (u + 1) % R

            @pl.when(c + 1 < nchunk)
            def _():
                @pl.when(c >= R - 1)
                def _():
                    pltpu.make_async_copy(buf.at[t1],
                                          out_hbm.at[pl.ds(base, C)],
                                          outs[t1]).wait()
                issue_in(c + 1, t1)

            def drain(i, _):
                pltpu.make_async_copy(a_hbm.at[pl.ds(base, 1)],
                                      abuf.at[q, pl.ds(0, 1)], gss[q]).wait()
                return _

            lax.fori_loop(0, k, drain, 0)

            def patch(i, _):
                g = _extract(idxv, s + i)
                p = g - (base + c * C)

                def add_body(d, c2):
                    slc = pl.ds(d * L, L)
                    plsc.addupdate(buf.at[t, p, slc], abuf[q, i, slc])
                    return c2

                lax.fori_loop(0, D // L, add_body, 0, unroll=16)
                return _

            lax.fori_loop(0, k, patch, 0)
            e2 = _extract(stv, c + 2)
            fire_rows(e, e2, 1 - q)
            pltpu.async_copy(buf.at[t], out_hbm.at[pl.ds(base + c * C, C)],
                             outs[t])
            return (e, e2)

        def group_body(gi, carry):
            for u in range(R):
                carry = chunk_work(gi * R + u, u, carry)
            return carry

        lax.fori_loop(0, nchunk // R, group_body, (jnp.int32(0), e0))
        for t in range(R):
            pltpu.make_async_copy(buf.at[t], out_hbm.at[pl.ds(base, C)],
                                  outs[t]).wait()

    fn = pl.kernel(
        body,
        out_type=jax.ShapeDtypeStruct((N, D), jnp.float32),
        mesh=plsc.VectorSubcoreMesh(
            core_axis_name="c", subcore_axis_name="s",
            num_cores=NC, num_subcores=NS),
        scratch_types=[
            pltpu.VMEM((R, C, D), jnp.float32),
            pltpu.VMEM((2, C, D), jnp.float32),
            pltpu.VMEM((RW,), jnp.int32),
            pltpu.VMEM((96,), jnp.int32),
        ] + [pltpu.SemaphoreType.DMA] * 10,
        compiler_params=pltpu.CompilerParams(needs_layout_passes=False),
    )
    return fn(x2, a2, gidx, starts)


K_SC = 8192


def _tc_body(x_ref, a_ref, m_ref, o_ref):
    m = m_ref[...]
    o_ref[...] = jnp.where(m != 0.0, x_ref[...] + a_ref[...], x_ref[...])


def _tc_where(x2, a2, mf, K, N, D):
    RB = 256
    off = K // RB
    return pl.pallas_call(
        _tc_body,
        grid=((N - K) // RB,),
        in_specs=[
            pl.BlockSpec((RB, D), lambda i: (i + off, 0)),
            pl.BlockSpec((RB, D), lambda i: (i + off, 0)),
            pl.BlockSpec((RB, 1), lambda i: (i + off, 0)),
        ],
        out_specs=pl.BlockSpec((RB, D), lambda i: (i, 0)),
        out_shape=jax.ShapeDtypeStruct((N - K, D), jnp.float32),
    )(x2, a2, mf)


def kernel(x, attack, attack_mask):
    B, S, D = x.shape
    N = B * S
    K = K_SC
    RW = K // NW
    x2 = x.reshape(N, D)
    a2 = attack.astype(x.dtype).reshape(N, D)
    m = attack_mask[:, :S].reshape(N)
    m2 = m[:K].reshape(NW, RW)
    loc = jnp.argsort(~m2, axis=1, stable=True).astype(jnp.int32)
    gidx = loc + (jnp.arange(NW, dtype=jnp.int32) * RW)[:, None]
    ccnt = m2.reshape(NW, RW // C, C).sum(-1).astype(jnp.int32)
    starts = jnp.concatenate(
        [jnp.zeros((NW, 1), jnp.int32), jnp.cumsum(ccnt, axis=1)], axis=1)
    starts = jnp.pad(starts, ((0, 0), (0, 96 - starts.shape[1])))
    sc_out = _sc_masked_add(x2, a2, gidx, starts, K, D)
    if K < N:
        mf = m.astype(jnp.float32)[:, None]
        tc_out = _tc_where(x2, a2, mf, K, N, D)
        out2 = jnp.concatenate([sc_out, tc_out], axis=0)
    else:
        out2 = sc_out
    return out2.reshape(B, S, D)
